# Initial kernel scaffold; baseline (speedup 1.0000x reference)
import jax
import jax.numpy as jnp
from jax.experimental import pallas as pl


def kernel(x, edge_index, W1, b1, W2, b2):
    raise NotImplementedError("write your pallas kernel here")


# speedup vs baseline: 3.1874x; 3.1874x over previous
import functools

import jax
import jax.numpy as jnp
from jax import lax
from jax.experimental import pallas as pl
from jax.experimental.pallas import tpu as pltpu
from jax.experimental.pallas import tpu_sc as plsc

N = 10000
D = 128
E = 320000
NC = 2
NS = 16
NW = NC * NS
CH = 80
NCHUNK = 128
EPW = NCHUNK * CH
EPAD = NW * EPW
RPOOL = 10112
RPT = RPOOL // NS

@functools.cache
def _make_sc_pool():
    mesh = plsc.VectorSubcoreMesh(core_axis_name="c", subcore_axis_name="s")
    return functools.partial(
        pl.kernel,
        mesh=mesh,
        out_type=jax.ShapeDtypeStruct((NC, RPOOL, D), jnp.float32),
        scratch_types=[
            pltpu.VMEM((2, CH), jnp.int32),
            pltpu.VMEM((2, CH), jnp.int32),
            pltpu.VMEM((2, CH, D), jnp.float32),
            pltpu.VMEM_SHARED((RPOOL, D), jnp.float32),
            pltpu.SemaphoreType.DMA,
            pltpu.SemaphoreType.DMA,
        ],
    )(_sc_pool_body)


def _sc_pool_body(feat_hbm, src_hbm, dst_hbm, zeros_hbm, out_hbm,
                  src_v, dst_v, rows_v, pool_sh, isem, gsem):
    c = lax.axis_index("c")
    s = lax.axis_index("s")
    wid = c * NS + s
    pltpu.sync_copy(zeros_hbm.at[pl.ds(s * RPT, RPT)],
                    pool_sh.at[pl.ds(s * RPT, RPT)])
    pltpu.sync_copy(src_hbm.at[wid, 0], src_v.at[0])
    pltpu.sync_copy(dst_hbm.at[wid, 0], dst_v.at[0])
    pltpu.async_copy(src_hbm.at[wid, 1], src_v.at[1], isem)
    pltpu.async_copy(dst_hbm.at[wid, 1], dst_v.at[1], isem)
    pltpu.async_copy(feat_hbm.at[src_v.at[0]], rows_v.at[0], gsem)
    plsc.subcore_barrier()

    def body(j, carry):
        slot = lax.rem(j, 2)
        nslot = lax.rem(j + 1, 2)

        @pl.when(j + 1 < NCHUNK)
        def _():
            pltpu.make_async_copy(src_hbm.at[wid, 0], src_v.at[0], isem).wait()
            pltpu.make_async_copy(dst_hbm.at[wid, 0], dst_v.at[0], isem).wait()
            pltpu.async_copy(feat_hbm.at[src_v.at[nslot]], rows_v.at[nslot],
                             gsem)

        pltpu.make_async_copy(feat_hbm.at[src_v.at[0]], rows_v.at[0],
                              gsem).wait()
        pltpu.sync_copy(rows_v.at[slot], pool_sh.at[dst_v.at[slot]], add=True)

        @pl.when(j + 2 < NCHUNK)
        def _():
            pltpu.async_copy(src_hbm.at[wid, j + 2], src_v.at[slot], isem)
            pltpu.async_copy(dst_hbm.at[wid, j + 2], dst_v.at[slot], isem)
        return carry

    lax.fori_loop(0, NCHUNK, body, 0)
    plsc.subcore_barrier()
    pltpu.sync_copy(pool_sh.at[pl.ds(s * RPT, RPT)],
                    out_hbm.at[c, pl.ds(s * RPT, RPT)])


def _dense_body(f_ref, pa_ref, pb_ref, wa_ref, wb_ref, b_ref, o_ref):
    f = f_ref[...]
    p = pa_ref[0] + pb_ref[0]
    acc = jnp.dot(f + p, wa_ref[...], preferred_element_type=jnp.float32,
                  precision=lax.Precision.HIGHEST)
    acc = acc + jnp.dot(f * p, wb_ref[...], preferred_element_type=jnp.float32,
                        precision=lax.Precision.HIGHEST)
    o_ref[...] = jnp.maximum(acc + b_ref[...], 0.0)


def _dense_norm_body(f_ref, pa_ref, pb_ref, wa_ref, wb_ref, b_ref, o_ref):
    f = f_ref[...]
    p = pa_ref[0] + pb_ref[0]
    acc = jnp.dot(f + p, wa_ref[...], preferred_element_type=jnp.float32,
                  precision=lax.Precision.HIGHEST)
    acc = acc + jnp.dot(f * p, wb_ref[...], preferred_element_type=jnp.float32,
                        precision=lax.Precision.HIGHEST)
    h = jnp.maximum(acc + b_ref[...], 0.0)
    nrm = jnp.sqrt(jnp.sum(h * h, axis=-1, keepdims=True))
    o_ref[...] = h / jnp.maximum(nrm, 1e-12)


def _dense(feat, parts, wa, wb, brow, normalize):
    body = _dense_norm_body if normalize else _dense_body
    return pl.pallas_call(
        body,
        grid=(1,),
        out_shape=jax.ShapeDtypeStruct((N, D), jnp.float32),
        in_specs=[
            pl.BlockSpec((N, D), lambda i: (0, 0)),
            pl.BlockSpec((1, N, D), lambda i: (0, 0, 0)),
            pl.BlockSpec((1, N, D), lambda i: (1, 0, 0)),
            pl.BlockSpec((D, D), lambda i: (0, 0)),
            pl.BlockSpec((D, D), lambda i: (0, 0)),
            pl.BlockSpec((1, D), lambda i: (0, 0)),
        ],
        out_specs=pl.BlockSpec((N, D), lambda i: (0, 0)),
    )(feat, parts, parts, wa, wb, brow)


def kernel(x, edge_index, W1, b1, W2, b2):
    src = edge_index[0]
    dst = edge_index[1]
    pad = EPAD - E
    srcp = jnp.concatenate([src, jnp.zeros((pad,), jnp.int32)])
    srcp = srcp.reshape(NW, NCHUNK, CH)
    dstp = jnp.concatenate([dst, jnp.full((pad,), N, jnp.int32)])
    dstp = dstp.reshape(NW, NCHUNK, CH)
    zeros = jnp.zeros((RPOOL, D), jnp.float32)

    w1a = W1[:, :D].T
    w1b = W1[:, D:].T
    w2a = W2[:, :D].T
    w2b = W2[:, D:].T

    sc_pool = _make_sc_pool()
    parts1 = sc_pool(x, srcp, dstp, zeros)
    h1 = _dense(x, parts1, w1a, w1b, b1.reshape(1, D), normalize=False)
    parts2 = sc_pool(h1, srcp, dstp, zeros)
    return _dense(h1, parts2, w2a, w2b, b2.reshape(1, D), normalize=True)
